# baseline (speedup 1.0000x reference)
import functools

import jax
import jax.numpy as jnp
from jax import lax
from jax.experimental import pallas as pl
from jax.experimental.pallas import tpu as pltpu
from jax.experimental.pallas import tpu_sc as plsc

B, S = 1024, 200
VOCAB, C, D = 100000, 4, 64
CDW = C * D // 2
DW = D // 2
WIN = 104
OUT_PER_TASK = 100
TASKS = B * 2
NC, NS = 2, 16
NW = NC * NS
TASKS_PER_W = TASKS // NW

_HI = -65536
_RND = 0x8000


def _split(w):
    fe = lax.bitcast_convert_type(lax.shift_left(w, 16), jnp.float32)
    fo = lax.bitcast_convert_type(lax.bitwise_and(w, _HI), jnp.float32)
    return fe, fo


def _repack(fe, fo):
    be = lax.shift_right_logical(
        lax.bitcast_convert_type(fe, jnp.int32) + _RND, 16)
    bo = lax.bitwise_and(lax.bitcast_convert_type(fo, jnp.int32) + _RND, _HI)
    return lax.bitwise_or(bo, be)


def _compute(emb, outb):
    def s_body(s, _):
        for g in range(DW // 16):
            ae, ao = _split(emb[s, pl.ds(g * 16, 16)])
            for i in range(1, C):
                fe, fo = _split(emb[s + i, pl.ds(i * DW + g * 16, 16)])
                ae = ae + fe
                ao = ao + fo
            outb[s, pl.ds(g * 16, 16)] = _repack(ae, ao)
        return 0

    lax.fori_loop(0, OUT_PER_TASK, s_body, 0, unroll=2)


def _sc_body(x2win_hbm, table_hbm, out_hbm, idx_all,
             emb0, emb1, out0, out1, sg0, sg1, so0, so1):
    wid = lax.axis_index("s") * NC + lax.axis_index("c")
    base_task = wid * TASKS_PER_W
    pltpu.sync_copy(x2win_hbm.at[pl.ds(base_task, TASKS_PER_W)], idx_all)

    def gather(t, embb, sem):
        return pltpu.async_copy(table_hbm.at[idx_all.at[t]], embb, sem)

    def gather_wait(t, embb, sem):
        pltpu.make_async_copy(table_hbm.at[idx_all.at[t]], embb, sem).wait()

    def scatter(t, outb, sem):
        return pltpu.async_copy(outb, out_hbm.at[base_task + t], sem)

    def scatter_wait(t, outb, sem):
        pltpu.make_async_copy(outb, out_hbm.at[base_task + t], sem).wait()

    gather(0, emb0, sg0)

    def task_body(k, _):
        t0 = 2 * k
        gather(t0 + 1, emb1, sg1)
        gather_wait(t0, emb0, sg0)

        @pl.when(k >= 1)
        def _():
            scatter_wait(t0 - 2, out0, so0)

        _compute(emb0, out0)
        scatter(t0, out0, so0)

        @pl.when(k < TASKS_PER_W // 2 - 1)
        def _():
            gather(t0 + 2, emb0, sg0)

        gather_wait(t0 + 1, emb1, sg1)

        @pl.when(k >= 1)
        def _():
            scatter_wait(t0 - 1, out1, so1)

        _compute(emb1, out1)
        scatter(t0 + 1, out1, so1)
        return 0

    lax.fori_loop(0, TASKS_PER_W // 2, task_body, 0)
    scatter_wait(TASKS_PER_W - 2, out0, so0)
    scatter_wait(TASKS_PER_W - 1, out1, so1)


def kernel(x, table):
    x = x.astype(jnp.int32)
    x2 = jnp.concatenate([x, jnp.zeros((B, C), jnp.int32)], axis=1)
    x2win = jnp.stack([x2[:, :WIN], x2[:, S - OUT_PER_TASK:]], axis=1)
    x2win = x2win.reshape(TASKS, WIN)
    table_bf = table.reshape(VOCAB, CDW, 2).astype(jnp.bfloat16)
    table_pk = lax.bitcast_convert_type(table_bf, jnp.int32)

    mesh = plsc.VectorSubcoreMesh(core_axis_name="c", subcore_axis_name="s")
    run = functools.partial(
        pl.kernel,
        mesh=mesh,
        out_type=jax.ShapeDtypeStruct((TASKS, OUT_PER_TASK, DW), jnp.int32),
        scratch_types=[
            pltpu.VMEM((TASKS_PER_W, WIN), jnp.int32),
            pltpu.VMEM((WIN, CDW), jnp.int32),
            pltpu.VMEM((WIN, CDW), jnp.int32),
            pltpu.VMEM((OUT_PER_TASK, DW), jnp.int32),
            pltpu.VMEM((OUT_PER_TASK, DW), jnp.int32),
            pltpu.SemaphoreType.DMA,
            pltpu.SemaphoreType.DMA,
            pltpu.SemaphoreType.DMA,
            pltpu.SemaphoreType.DMA,
        ],
    )(_sc_body)
    out = run(x2win, table_pk)
    out = lax.bitcast_convert_type(out, jnp.bfloat16)
    return out.astype(jnp.float32).reshape(B, S, D)


# speedup vs baseline: 1.4319x; 1.4319x over previous
import functools

import jax
import jax.numpy as jnp
from jax import lax
from jax.experimental import pallas as pl
from jax.experimental.pallas import tpu as pltpu
from jax.experimental.pallas import tpu_sc as plsc

B, S = 1024, 200
VOCAB, C, D = 100000, 4, 64
CDW = C * D // 2
WIN = 104
OUT_PER_TASK = 100
TASKS = B * 2
NC, NS = 2, 16
NW = NC * NS
TASKS_PER_W = TASKS // NW

_HI = -65536


def _lo(w):
    return lax.bitcast_convert_type(lax.shift_left(w, 16), jnp.float32)


def _hi(w):
    return lax.bitcast_convert_type(lax.bitwise_and(w, _HI), jnp.float32)


def _compute(emb, outb):
    def s_body(s, _):
        for g in range(D // 16):
            c = g * 16
            acc = _lo(emb[s, pl.ds(c, 16)])
            acc = acc + _lo(emb[s + 1, pl.ds(64 + c, 16)])
            acc = acc + _hi(emb[s + 2, pl.ds(c, 16)])
            acc = acc + _hi(emb[s + 3, pl.ds(64 + c, 16)])
            outb[s, pl.ds(c, 16)] = acc
        return 0

    lax.fori_loop(0, OUT_PER_TASK, s_body, 0, unroll=2)


def _sc_body(x2win_hbm, table_hbm, out_hbm, idx_all,
             emb0, emb1, out0, out1, sg0, sg1, so0, so1):
    wid = lax.axis_index("s") * NC + lax.axis_index("c")
    base_task = wid * TASKS_PER_W
    pltpu.sync_copy(x2win_hbm.at[pl.ds(base_task, TASKS_PER_W)], idx_all)

    def gather(t, embb, sem):
        return pltpu.async_copy(table_hbm.at[idx_all.at[t]], embb, sem)

    def gather_wait(t, embb, sem):
        pltpu.make_async_copy(table_hbm.at[idx_all.at[t]], embb, sem).wait()

    def scatter(t, outb, sem):
        return pltpu.async_copy(outb, out_hbm.at[base_task + t], sem)

    def scatter_wait(t, outb, sem):
        pltpu.make_async_copy(outb, out_hbm.at[base_task + t], sem).wait()

    gather(0, emb0, sg0)

    def task_body(k, _):
        t0 = 2 * k
        gather(t0 + 1, emb1, sg1)
        gather_wait(t0, emb0, sg0)

        @pl.when(k >= 1)
        def _():
            scatter_wait(t0 - 2, out0, so0)

        _compute(emb0, out0)
        scatter(t0, out0, so0)

        @pl.when(k < TASKS_PER_W // 2 - 1)
        def _():
            gather(t0 + 2, emb0, sg0)

        gather_wait(t0 + 1, emb1, sg1)

        @pl.when(k >= 1)
        def _():
            scatter_wait(t0 - 1, out1, so1)

        _compute(emb1, out1)
        scatter(t0 + 1, out1, so1)
        return 0

    lax.fori_loop(0, TASKS_PER_W // 2, task_body, 0)
    scatter_wait(TASKS_PER_W - 2, out0, so0)
    scatter_wait(TASKS_PER_W - 1, out1, so1)


def _pack_table(table):
    bits = lax.bitcast_convert_type(table.reshape(VOCAB, 2 * CDW), jnp.int32)
    lo = lax.shift_right_logical(bits[:, :CDW] + 0x8000, 16)
    hi = lax.bitwise_and(bits[:, CDW:] + 0x8000, _HI)
    return lax.bitwise_or(hi, lo)


def kernel(x, table):
    x = x.astype(jnp.int32)
    x2 = jnp.concatenate([x, jnp.zeros((B, C), jnp.int32)], axis=1)
    x2win = jnp.stack([x2[:, :WIN], x2[:, S - OUT_PER_TASK:]], axis=1)
    x2win = x2win.reshape(TASKS, WIN)
    table_pk = _pack_table(table)

    mesh = plsc.VectorSubcoreMesh(core_axis_name="c", subcore_axis_name="s")
    run = functools.partial(
        pl.kernel,
        mesh=mesh,
        out_type=jax.ShapeDtypeStruct((TASKS, OUT_PER_TASK, D), jnp.float32),
        scratch_types=[
            pltpu.VMEM((TASKS_PER_W, WIN), jnp.int32),
            pltpu.VMEM((WIN, CDW), jnp.int32),
            pltpu.VMEM((WIN, CDW), jnp.int32),
            pltpu.VMEM((OUT_PER_TASK, D), jnp.float32),
            pltpu.VMEM((OUT_PER_TASK, D), jnp.float32),
            pltpu.SemaphoreType.DMA,
            pltpu.SemaphoreType.DMA,
            pltpu.SemaphoreType.DMA,
            pltpu.SemaphoreType.DMA,
        ],
    )(_sc_body)
    out = run(x2win, table_pk)
    return out.reshape(B, S, D)


# speedup vs baseline: 2.2219x; 1.5517x over previous
import functools

import jax
import jax.numpy as jnp
from jax import lax
from jax.experimental import pallas as pl
from jax.experimental.pallas import tpu as pltpu
from jax.experimental.pallas import tpu_sc as plsc

B, S = 1024, 200
VOCAB, C, D = 100000, 4, 64
CD = C * D
WIN = 104
HS = S // 2
NC, NS = 2, 16
NW = NC * NS
TASKS_PER_W = B // NW


def _acc_row(loads):
    acc = loads[0]
    for v in loads[1:]:
        acc = acc + v
    return acc


def _compute(embA, embB, outb):
    def s_lo(r, _):
        for h in range(2):
            for g in range(D // 16):
                outb[r, pl.ds(h * D + g * 16, 16)] = _acc_row(
                    [embA[2 * r + h + i, pl.ds(i * D + g * 16, 16)]
                     for i in range(C)])
        return 0

    lax.fori_loop(0, HS // 2, s_lo, 0, unroll=2)

    def s_hi(r, _):
        for h in range(2):
            for g in range(D // 16):
                outb[r, pl.ds(h * D + g * 16, 16)] = _acc_row(
                    [embB[2 * r + h + i - HS, pl.ds(i * D + g * 16, 16)]
                     for i in range(C)])
        return 0

    lax.fori_loop(HS // 2, S // 2, s_hi, 0, unroll=2)


def _sc_body(x2_hbm, table_hbm, out_hbm, idx0, idx1,
             embA0, embB0, embA1, embB1, outb,
             si0, si1, sg0, sg1, so):
    wid = lax.axis_index("s") * NC + lax.axis_index("c")
    base = wid * TASKS_PER_W

    def idx_fetch(t, idxb, sem):
        return pltpu.async_copy(x2_hbm.at[base + t], idxb, sem)

    def idx_wait(t, idxb, sem):
        pltpu.make_async_copy(x2_hbm.at[base + t], idxb, sem).wait()

    def gather(idxb, embAb, embBb, sem):
        pltpu.async_copy(table_hbm.at[idxb.at[0]], embAb, sem)
        pltpu.async_copy(table_hbm.at[idxb.at[1]], embBb, sem)

    def gather_wait(idxb, embAb, embBb, sem):
        pltpu.make_async_copy(table_hbm.at[idxb.at[0]], embAb, sem).wait()
        pltpu.make_async_copy(table_hbm.at[idxb.at[1]], embBb, sem).wait()

    def scatter(t, sem):
        return pltpu.async_copy(outb, out_hbm.at[base + t], sem)

    def scatter_wait(t, sem):
        pltpu.make_async_copy(outb, out_hbm.at[base + t], sem).wait()

    pltpu.sync_copy(x2_hbm.at[base], idx0)
    gather(idx0, embA0, embB0, sg0)
    idx_fetch(1, idx1, si1)

    def task_body(k, _):
        t0 = 2 * k
        idx_wait(t0 + 1, idx1, si1)
        gather(idx1, embA1, embB1, sg1)
        gather_wait(idx0, embA0, embB0, sg0)

        @pl.when(k < TASKS_PER_W // 2 - 1)
        def _():
            idx_fetch(t0 + 2, idx0, si0)

        @pl.when(k >= 1)
        def _():
            scatter_wait(t0 - 1, so)

        _compute(embA0, embB0, outb)
        scatter(t0, so)

        @pl.when(k < TASKS_PER_W // 2 - 1)
        def _():
            idx_wait(t0 + 2, idx0, si0)
            gather(idx0, embA0, embB0, sg0)

        gather_wait(idx1, embA1, embB1, sg1)

        @pl.when(k < TASKS_PER_W // 2 - 1)
        def _():
            idx_fetch(t0 + 3, idx1, si1)

        scatter_wait(t0, so)
        _compute(embA1, embB1, outb)
        scatter(t0 + 1, so)
        return 0

    lax.fori_loop(0, TASKS_PER_W // 2, task_body, 0)
    scatter_wait(TASKS_PER_W - 1, so)


def kernel(x, table):
    x = x.astype(jnp.int32)
    x2 = jnp.concatenate([x, jnp.zeros((B, C), jnp.int32)], axis=1)
    x2win = jnp.stack([x2[:, :WIN], x2[:, S - HS:]], axis=1)
    table2d = table.reshape(VOCAB, CD)

    mesh = plsc.VectorSubcoreMesh(core_axis_name="c", subcore_axis_name="s")
    run = functools.partial(
        pl.kernel,
        mesh=mesh,
        out_type=jax.ShapeDtypeStruct((B, S // 2, 2 * D), jnp.float32),
        scratch_types=[
            pltpu.VMEM((2, WIN), jnp.int32),
            pltpu.VMEM((2, WIN), jnp.int32),
            pltpu.VMEM((WIN, CD), jnp.float32),
            pltpu.VMEM((WIN, CD), jnp.float32),
            pltpu.VMEM((WIN, CD), jnp.float32),
            pltpu.VMEM((WIN, CD), jnp.float32),
            pltpu.VMEM((S // 2, 2 * D), jnp.float32),
            pltpu.SemaphoreType.DMA,
            pltpu.SemaphoreType.DMA,
            pltpu.SemaphoreType.DMA,
            pltpu.SemaphoreType.DMA,
            pltpu.SemaphoreType.DMA,
        ],
    )(_sc_body)
    return run(x2win, table2d).reshape(B, S, D)
